# Initial kernel scaffold; baseline (speedup 1.0000x reference)
import jax
import jax.numpy as jnp
from jax.experimental import pallas as pl


def kernel(x, emb):
    raise NotImplementedError("write your pallas kernel here")


# speedup vs baseline: 1.7546x; 1.7546x over previous
import functools

import jax
import jax.numpy as jnp
from jax import lax
from jax.experimental import pallas as pl
from jax.experimental.pallas import tpu as pltpu
from jax.experimental.pallas import tpu_sc as plsc

_B, _L, _DX, _DE = 4096, 200, 64, 128
_DO = _DX + _DE
_NSLOT = 2


def kernel(x, emb):
    info = plsc.get_sparse_core_info()
    nw = info.num_cores * info.num_subcores
    b_per_w = _B // nw
    mesh = plsc.VectorSubcoreMesh(core_axis_name="c", subcore_axis_name="s")

    @functools.partial(
        pl.kernel,
        mesh=mesh,
        compiler_params=pltpu.CompilerParams(use_tc_tiling_on_sc=False),
        out_type=jax.ShapeDtypeStruct((_B, _L, _DO), jnp.float32),
        scratch_types=[
            pltpu.VMEM((_NSLOT, _L, _DO), jnp.float32),
            pltpu.SemaphoreType.DMA,
            pltpu.SemaphoreType.DMA,
            pltpu.SemaphoreType.DMA,
            pltpu.SemaphoreType.DMA,
        ],
    )
    def _run(x_hbm, emb_hbm, out_hbm, rows_v, in_sem0, in_sem1, out_sem0, out_sem1):
        in_sems = (in_sem0, in_sem1)
        out_sems = (out_sem0, out_sem1)
        wid = lax.axis_index("s") * info.num_cores + lax.axis_index("c")
        base = wid * b_per_w

        for s in range(_NSLOT):
            pltpu.sync_copy(
                emb_hbm.at[pl.ds(1, _L - 1)],
                rows_v.at[s, pl.ds(0, _L - 1), pl.ds(_DX, _DE)],
            )
            pltpu.sync_copy(
                emb_hbm.at[pl.ds(0, 1)],
                rows_v.at[s, pl.ds(_L - 1, 1), pl.ds(_DX, _DE)],
            )

        def in_copy(r, s):
            return pltpu.make_async_copy(
                x_hbm.at[base + r], rows_v.at[s, :, pl.ds(0, _DX)], in_sems[s]
            )

        def out_copy(r, s):
            return pltpu.make_async_copy(
                rows_v.at[s], out_hbm.at[base + r], out_sems[s]
            )

        in_copy(0, 0).start()

        def body(i, carry):
            for j in range(_NSLOT):
                r = i * _NSLOT + j
                in_copy(r, j).wait()
                out_copy(r, j).start()
                rn = r + 1
                sn = (j + 1) % _NSLOT

                @pl.when(rn < b_per_w)
                def _prefetch():
                    @pl.when(rn >= _NSLOT)
                    def _slot_free():
                        out_copy(rn - _NSLOT, sn).wait()

                    in_copy(rn, sn).start()

            return carry

        lax.fori_loop(0, b_per_w // _NSLOT, body, 0)
        out_copy(b_per_w - 2, 0).wait()
        out_copy(b_per_w - 1, 1).wait()

    return _run(x, emb)


# speedup vs baseline: 1.8881x; 1.0761x over previous
import functools

import jax
import jax.numpy as jnp
from jax import lax
from jax.experimental import pallas as pl
from jax.experimental.pallas import tpu as pltpu
from jax.experimental.pallas import tpu_sc as plsc

_B, _L, _DX, _DE = 4096, 200, 64, 128
_DO = _DX + _DE
_NSLOT = 2
_LANES = 16


def kernel(x, emb):
    info = plsc.get_sparse_core_info()
    nw = info.num_cores * info.num_subcores
    b_per_w = _B // nw
    mesh = plsc.VectorSubcoreMesh(core_axis_name="c", subcore_axis_name="s")

    @functools.partial(
        pl.kernel,
        mesh=mesh,
        compiler_params=pltpu.CompilerParams(use_tc_tiling_on_sc=False),
        out_type=jax.ShapeDtypeStruct((_B, _L * _DO), jnp.float32),
        scratch_types=[
            pltpu.VMEM((_NSLOT, _L * _DO), jnp.float32),
            pltpu.VMEM((_NSLOT, _L * _DX), jnp.float32),
            pltpu.VMEM((_L * _DE,), jnp.float32),
            pltpu.SemaphoreType.DMA,
            pltpu.SemaphoreType.DMA,
            pltpu.SemaphoreType.DMA,
            pltpu.SemaphoreType.DMA,
        ],
    )
    def _run(x_hbm, emb_hbm, out_hbm, rows_v, xbuf_v, tbl_v,
             in_sem0, in_sem1, out_sem0, out_sem1):
        in_sems = (in_sem0, in_sem1)
        out_sems = (out_sem0, out_sem1)
        wid = lax.axis_index("s") * info.num_cores + lax.axis_index("c")
        base = wid * b_per_w

        pltpu.sync_copy(emb_hbm.at[pl.ds(_DE, (_L - 1) * _DE)],
                        tbl_v.at[pl.ds(0, (_L - 1) * _DE)])
        pltpu.sync_copy(emb_hbm.at[pl.ds(0, _DE)],
                        tbl_v.at[pl.ds((_L - 1) * _DE, _DE)])

        def fill(l, carry):
            for s in range(_NSLOT):
                for k in range(_DE // _LANES):
                    rows_v[s, pl.ds(l * _DO + _DX + k * _LANES, _LANES)] = (
                        tbl_v[pl.ds(l * _DE + k * _LANES, _LANES)])
            return carry

        lax.fori_loop(0, _L, fill, 0)

        def in_copy(r, s):
            return pltpu.make_async_copy(
                x_hbm.at[base + r], xbuf_v.at[s], in_sems[s])

        def out_copy(r, s):
            return pltpu.make_async_copy(
                rows_v.at[s], out_hbm.at[base + r], out_sems[s])

        in_copy(0, 0).start()

        def iter_body(i, carry):
            for j in range(_NSLOT):
                r = i * _NSLOT + j
                in_copy(r, j).wait()

                def ilv(l, c):
                    for u in range(2):
                        for k in range(_DX // _LANES):
                            rows_v[j, pl.ds((2 * l + u) * _DO + k * _LANES,
                                            _LANES)] = (
                                xbuf_v[j, pl.ds((2 * l + u) * _DX + k * _LANES,
                                                _LANES)])
                    return c

                lax.fori_loop(0, _L // 2, ilv, 0)
                out_copy(r, j).start()
                rn = r + 1

                @pl.when(rn < b_per_w)
                def _prefetch():
                    in_copy(rn, (j + 1) % _NSLOT).start()

                @pl.when(r >= 1)
                def _drain_prev():
                    out_copy(r - 1, (j + 1) % _NSLOT).wait()

            return carry

        lax.fori_loop(0, b_per_w // _NSLOT, iter_body, 0)
        out_copy(b_per_w - 1, (b_per_w - 1) % _NSLOT).wait()

    return _run(x.reshape(_B, _L * _DX),
                emb.reshape(-1)).reshape(_B, _L, _DO)


# speedup vs baseline: 2.0794x; 1.1014x over previous
import functools

import jax
import jax.numpy as jnp
from jax import lax
from jax.experimental import pallas as pl
from jax.experimental.pallas import tpu as pltpu
from jax.experimental.pallas import tpu_sc as plsc

_B, _L, _DX, _DE = 4096, 200, 64, 128
_DO = _DX + _DE
_NROW = 2
_NIN = 4
_LANES = 16
_XW = _L * _DX
_OW = _L * _DO
_EW = _L * _DE


def kernel(x, emb):
    info = plsc.get_sparse_core_info()
    nw = info.num_cores * info.num_subcores
    b_per_w = _B // nw
    mesh = plsc.VectorSubcoreMesh(core_axis_name="c", subcore_axis_name="s")

    @functools.partial(
        pl.kernel,
        mesh=mesh,
        compiler_params=pltpu.CompilerParams(use_tc_tiling_on_sc=False),
        out_type=jax.ShapeDtypeStruct((_B, _OW), jnp.float32),
        scratch_types=[
            pltpu.VMEM((_NROW, _OW), jnp.float32),
            pltpu.VMEM((_NIN * _XW,), jnp.float32),
            pltpu.SemaphoreType.DMA,
            pltpu.SemaphoreType.DMA,
            pltpu.SemaphoreType.DMA,
            pltpu.SemaphoreType.DMA,
            pltpu.SemaphoreType.DMA,
            pltpu.SemaphoreType.DMA,
        ],
    )
    def _run(x_hbm, emb_hbm, out_hbm, rows_v, xbuf_v,
             in_sem0, in_sem1, in_sem2, in_sem3, out_sem0, out_sem1):
        in_sems = (in_sem0, in_sem1, in_sem2, in_sem3)
        out_sems = (out_sem0, out_sem1)
        wid = lax.axis_index("s") * info.num_cores + lax.axis_index("c")
        base = wid * b_per_w

        pltpu.sync_copy(emb_hbm.at[pl.ds(_DE, _EW - _DE)],
                        xbuf_v.at[pl.ds(0, _EW - _DE)])
        pltpu.sync_copy(emb_hbm.at[pl.ds(0, _DE)],
                        xbuf_v.at[pl.ds(_EW - _DE, _DE)])

        def fill(l, carry):
            for s in range(_NROW):
                for k in range(_DE // _LANES):
                    rows_v[s, pl.ds(l * _DO + _DX + k * _LANES, _LANES)] = (
                        xbuf_v[pl.ds(l * _DE + k * _LANES, _LANES)])
            return carry

        lax.fori_loop(0, _L, fill, 0)

        def in_copy(r, q):
            return pltpu.make_async_copy(
                x_hbm.at[base + r], xbuf_v.at[pl.ds(q * _XW, _XW)], in_sems[q])

        def out_copy(r, s):
            return pltpu.make_async_copy(
                rows_v.at[s], out_hbm.at[base + r], out_sems[s])

        in_copy(0, 0).start()
        in_copy(1, 1).start()

        def iter_body(i, carry):
            for u in range(_NIN):
                r = i * _NIN + u
                j = u % _NROW
                in_copy(r, u).wait()

                def ilv(l, c):
                    for v in range(2):
                        for k in range(_DX // _LANES):
                            rows_v[j, pl.ds((2 * l + v) * _DO + k * _LANES,
                                            _LANES)] = (
                                xbuf_v[pl.ds(u * _XW + (2 * l + v) * _DX
                                             + k * _LANES, _LANES)])
                    return c

                lax.fori_loop(0, _L // 2, ilv, 0)
                out_copy(r, j).start()
                rn = r + 2

                @pl.when(rn < b_per_w)
                def _prefetch():
                    in_copy(rn, (u + 2) % _NIN).start()

                @pl.when(r >= 1)
                def _drain_prev():
                    out_copy(r - 1, (j + 1) % _NROW).wait()

            return carry

        lax.fori_loop(0, b_per_w // _NIN, iter_body, 0)
        out_copy(b_per_w - 1, (b_per_w - 1) % _NROW).wait()

    return _run(x.reshape(_B, _XW), emb.reshape(-1)).reshape(_B, _L, _DO)


# speedup vs baseline: 2.1517x; 1.0348x over previous
import functools

import jax
import jax.numpy as jnp
from jax import lax
from jax.experimental import pallas as pl
from jax.experimental.pallas import tpu as pltpu
from jax.experimental.pallas import tpu_sc as plsc

_B, _L, _DX, _DE = 4096, 200, 64, 128
_DO = _DX + _DE
_NROW = 2
_NIN = 4
_LANES = 16
_XW = _L * _DX
_OW = _L * _DO
_EW = _L * _DE


def kernel(x, emb):
    info = plsc.get_sparse_core_info()
    nw = info.num_cores * info.num_subcores
    b_per_w = _B // nw
    mesh = plsc.VectorSubcoreMesh(core_axis_name="c", subcore_axis_name="s")

    @functools.partial(
        pl.kernel,
        mesh=mesh,
        compiler_params=pltpu.CompilerParams(use_tc_tiling_on_sc=False),
        out_type=jax.ShapeDtypeStruct((_B, _OW), jnp.float32),
        scratch_types=[
            pltpu.VMEM((_NROW, _OW), jnp.float32),
            pltpu.VMEM((_NIN * _XW,), jnp.float32),
            pltpu.SemaphoreType.DMA,
            pltpu.SemaphoreType.DMA,
            pltpu.SemaphoreType.DMA,
            pltpu.SemaphoreType.DMA,
            pltpu.SemaphoreType.DMA,
            pltpu.SemaphoreType.DMA,
        ],
    )
    def _run(x_hbm, emb_hbm, out_hbm, rows_v, xbuf_v,
             in_sem0, in_sem1, in_sem2, in_sem3, out_sem0, out_sem1):
        in_sems = (in_sem0, in_sem1, in_sem2, in_sem3)
        out_sems = (out_sem0, out_sem1)
        wid = lax.axis_index("s") * info.num_cores + lax.axis_index("c")
        base = wid * b_per_w

        pltpu.sync_copy(emb_hbm.at[pl.ds(_DE, _EW - _DE)],
                        xbuf_v.at[pl.ds(0, _EW - _DE)])
        pltpu.sync_copy(emb_hbm.at[pl.ds(0, _DE)],
                        xbuf_v.at[pl.ds(_EW - _DE, _DE)])

        @plsc.parallel_loop(0, _L, unroll=4)
        def fill(l):
            for s in range(_NROW):
                for k in range(_DE // _LANES):
                    rows_v[s, pl.ds(l * _DO + _DX + k * _LANES, _LANES)] = (
                        xbuf_v[pl.ds(l * _DE + k * _LANES, _LANES)])

        def in_copy(r, q):
            return pltpu.make_async_copy(
                x_hbm.at[base + r], xbuf_v.at[pl.ds(q * _XW, _XW)], in_sems[q])

        def out_copy(r, s):
            return pltpu.make_async_copy(
                rows_v.at[s], out_hbm.at[base + r], out_sems[s])

        in_copy(0, 0).start()
        in_copy(1, 1).start()

        def iter_body(i, carry):
            for u in range(_NIN):
                r = i * _NIN + u
                j = u % _NROW
                in_copy(r, u).wait()

                @plsc.parallel_loop(0, _L, unroll=8)
                def ilv(l):
                    for k in range(_DX // _LANES):
                        rows_v[j, pl.ds(l * _DO + k * _LANES, _LANES)] = (
                            xbuf_v[pl.ds(u * _XW + l * _DX + k * _LANES,
                                         _LANES)])
                out_copy(r, j).start()
                rn = r + 2

                @pl.when(rn < b_per_w)
                def _prefetch():
                    in_copy(rn, (u + 2) % _NIN).start()

                @pl.when(r >= 1)
                def _drain_prev():
                    out_copy(r - 1, (j + 1) % _NROW).wait()

            return carry

        lax.fori_loop(0, b_per_w // _NIN, iter_body, 0)
        out_copy(b_per_w - 1, (b_per_w - 1) % _NROW).wait()

    return _run(x.reshape(_B, _XW), emb.reshape(-1)).reshape(_B, _L, _DO)


# speedup vs baseline: 2.6193x; 1.2173x over previous
import functools

import jax
import jax.numpy as jnp
from jax import lax
from jax.experimental import pallas as pl
from jax.experimental.pallas import tpu as pltpu
from jax.experimental.pallas import tpu_sc as plsc

_B, _L, _DX, _DE = 4096, 200, 64, 128
_DO = _DX + _DE
_LANES = 16
_LPAD = 208
_MAXLEN = 200


def _sc_lookup(emb):
    mesh = plsc.VectorSubcoreMesh(core_axis_name="c", subcore_axis_name="s")

    @functools.partial(
        pl.kernel,
        mesh=mesh,
        compiler_params=pltpu.CompilerParams(use_tc_tiling_on_sc=False),
        out_type=jax.ShapeDtypeStruct((_LPAD, _DE), jnp.float32),
        scratch_types=[
            pltpu.VMEM((_LPAD,), jnp.int32),
            pltpu.VMEM((_LPAD, _DE), jnp.float32),
            pltpu.SemaphoreType.DMA,
        ],
    )
    def _run(emb_hbm, tbl_hbm, idx_v, rows_v, sem):
        wid = lax.axis_index("s") * 2 + lax.axis_index("c")

        @pl.when(wid == 0)
        def _():
            for c in range(_LPAD // _LANES):
                vals = lax.iota(jnp.int32, _LANES) + (c * _LANES + 1)
                idx_v[pl.ds(c * _LANES, _LANES)] = jnp.where(
                    vals < _MAXLEN, vals, 0)
            pltpu.async_copy(emb_hbm.at[idx_v], rows_v, sem).wait()
            pltpu.sync_copy(rows_v, tbl_hbm)

    return _run(emb)


def _tc_concat(x, tbl):
    bt = 32

    def body(x_ref, tbl_ref, o_ref):
        t = tbl_ref[pl.ds(0, _L), :]
        o_ref[...] = jnp.concatenate(
            [x_ref[...], jnp.broadcast_to(t[None], (bt, _L, _DE))], axis=2)

    return pl.pallas_call(
        body,
        grid=(_B // bt,),
        in_specs=[
            pl.BlockSpec((bt, _L, _DX), lambda i: (i, 0, 0)),
            pl.BlockSpec((_LPAD, _DE), lambda i: (0, 0)),
        ],
        out_specs=pl.BlockSpec((bt, _L, _DO), lambda i: (i, 0, 0)),
        out_shape=jax.ShapeDtypeStruct((_B, _L, _DO), jnp.float32),
    )(x, tbl)


def kernel(x, emb):
    return _tc_concat(x, _sc_lookup(emb))


# speedup vs baseline: 3.6137x; 1.3797x over previous
import functools

import jax
import jax.numpy as jnp
from jax import lax
from jax.experimental import pallas as pl
from jax.experimental.pallas import tpu as pltpu
from jax.experimental.pallas import tpu_sc as plsc

_B, _L, _DX, _DE = 4096, 200, 64, 128
_DO = _DX + _DE
_LANES = 16
_LPAD = 208
_MAXLEN = 200
_XW = _L * _DX
_OW = _L * _DO
_BT = 32


def _sc_lookup(emb):
    mesh = plsc.VectorSubcoreMesh(core_axis_name="c", subcore_axis_name="s")

    @functools.partial(
        pl.kernel,
        mesh=mesh,
        compiler_params=pltpu.CompilerParams(use_tc_tiling_on_sc=False),
        out_type=jax.ShapeDtypeStruct((_LPAD, _DE), jnp.float32),
        scratch_types=[
            pltpu.VMEM((_LPAD,), jnp.int32),
            pltpu.VMEM((_LPAD, _DE), jnp.float32),
            pltpu.SemaphoreType.DMA,
        ],
    )
    def _run(emb_hbm, tbl_hbm, idx_v, rows_v, sem):
        wid = lax.axis_index("s") * 2 + lax.axis_index("c")

        @pl.when(wid == 0)
        def _():
            for c in range(_LPAD // _LANES):
                vals = lax.iota(jnp.int32, _LANES) + (c * _LANES + 1)
                idx_v[pl.ds(c * _LANES, _LANES)] = jnp.where(
                    vals < _MAXLEN, vals, 0)
            pltpu.async_copy(emb_hbm.at[idx_v], rows_v, sem).wait()
            pltpu.sync_copy(rows_v, tbl_hbm)

    return _run(emb)


def _tc_concat(xf, tblf):
    def body(x_ref, tbl_ref, o_ref):
        for l in range(_L):
            o_ref[:, pl.ds(l * _DO, _DX)] = x_ref[:, pl.ds(l * _DX, _DX)]
            o_ref[:, pl.ds(l * _DO + _DX, _DE)] = jnp.broadcast_to(
                tbl_ref[pl.ds(l * _DE, _DE)][None], (_BT, _DE))

    return pl.pallas_call(
        body,
        grid=(_B // _BT,),
        in_specs=[
            pl.BlockSpec((_BT, _XW), lambda i: (i, 0)),
            pl.BlockSpec((_LPAD * _DE,), lambda i: (0,)),
        ],
        out_specs=pl.BlockSpec((_BT, _OW), lambda i: (i, 0)),
        out_shape=jax.ShapeDtypeStruct((_B, _OW), jnp.float32),
    )(xf, tblf)


def kernel(x, emb):
    tbl = _sc_lookup(emb)
    out = _tc_concat(x.reshape(_B, _XW), tbl.reshape(_LPAD * _DE))
    return out.reshape(_B, _L, _DO)


# speedup vs baseline: 3.6502x; 1.0101x over previous
import functools

import jax
import jax.numpy as jnp
from jax import lax
from jax.experimental import pallas as pl
from jax.experimental.pallas import tpu as pltpu
from jax.experimental.pallas import tpu_sc as plsc

_B, _L, _DX, _DE = 4096, 200, 64, 128
_DO = _DX + _DE
_LANES = 16
_LPAD = 208
_MAXLEN = 200
_XW = _L * _DX
_OW = _L * _DO
_BT = 64


def _sc_lookup(emb):
    mesh = plsc.VectorSubcoreMesh(core_axis_name="c", subcore_axis_name="s")

    @functools.partial(
        pl.kernel,
        mesh=mesh,
        compiler_params=pltpu.CompilerParams(use_tc_tiling_on_sc=False),
        out_type=jax.ShapeDtypeStruct((_LPAD, _DE), jnp.float32),
        scratch_types=[
            pltpu.VMEM((_LPAD,), jnp.int32),
            pltpu.VMEM((_LPAD, _DE), jnp.float32),
            pltpu.SemaphoreType.DMA,
        ],
    )
    def _run(emb_hbm, tbl_hbm, idx_v, rows_v, sem):
        wid = lax.axis_index("s") * 2 + lax.axis_index("c")

        @pl.when(wid == 0)
        def _():
            for c in range(_LPAD // _LANES):
                vals = lax.iota(jnp.int32, _LANES) + (c * _LANES + 1)
                idx_v[pl.ds(c * _LANES, _LANES)] = jnp.where(
                    vals < _MAXLEN, vals, 0)
            pltpu.async_copy(emb_hbm.at[idx_v], rows_v, sem).wait()
            pltpu.sync_copy(rows_v, tbl_hbm)

    return _run(emb)


def _tc_concat(xf, tblf):
    def body(x_ref, tbl_ref, o_ref):
        for l in range(_L):
            o_ref[:, pl.ds(l * _DO, _DX)] = x_ref[:, pl.ds(l * _DX, _DX)]
            o_ref[:, pl.ds(l * _DO + _DX, _DE)] = jnp.broadcast_to(
                tbl_ref[pl.ds(l * _DE, _DE)][None], (_BT, _DE))

    return pl.pallas_call(
        body,
        grid=(_B // _BT,),
        in_specs=[
            pl.BlockSpec((_BT, _XW), lambda i: (i, 0)),
            pl.BlockSpec((_LPAD * _DE,), lambda i: (0,)),
        ],
        out_specs=pl.BlockSpec((_BT, _OW), lambda i: (i, 0)),
        out_shape=jax.ShapeDtypeStruct((_B, _OW), jnp.float32),
    )(xf, tblf)


def kernel(x, emb):
    tbl = _sc_lookup(emb)
    out = _tc_concat(x.reshape(_B, _XW), tbl.reshape(_LPAD * _DE))
    return out.reshape(_B, _L, _DO)


# speedup vs baseline: 3.6781x; 1.0076x over previous
import functools

import jax
import jax.numpy as jnp
from jax import lax
from jax.experimental import pallas as pl
from jax.experimental.pallas import tpu as pltpu
from jax.experimental.pallas import tpu_sc as plsc

_B, _L, _DX, _DE = 4096, 200, 64, 128
_DO = _DX + _DE
_LANES = 16
_LPAD = 208
_MAXLEN = 200
_XW = _L * _DX
_OW = _L * _DO
_BT = 128


def _sc_lookup(emb):
    mesh = plsc.VectorSubcoreMesh(core_axis_name="c", subcore_axis_name="s")

    @functools.partial(
        pl.kernel,
        mesh=mesh,
        compiler_params=pltpu.CompilerParams(use_tc_tiling_on_sc=False),
        out_type=jax.ShapeDtypeStruct((_LPAD, _DE), jnp.float32),
        scratch_types=[
            pltpu.VMEM((_LPAD,), jnp.int32),
            pltpu.VMEM((_LPAD, _DE), jnp.float32),
            pltpu.SemaphoreType.DMA,
        ],
    )
    def _run(emb_hbm, tbl_hbm, idx_v, rows_v, sem):
        wid = lax.axis_index("s") * 2 + lax.axis_index("c")

        @pl.when(wid == 0)
        def _():
            for c in range(_LPAD // _LANES):
                vals = lax.iota(jnp.int32, _LANES) + (c * _LANES + 1)
                idx_v[pl.ds(c * _LANES, _LANES)] = jnp.where(
                    vals < _MAXLEN, vals, 0)
            pltpu.async_copy(emb_hbm.at[idx_v], rows_v, sem).wait()
            pltpu.sync_copy(rows_v, tbl_hbm)

    return _run(emb)


def _tc_concat(xf, tblf):
    def body(x_ref, tbl_ref, o_ref):
        for l in range(_L):
            o_ref[:, pl.ds(l * _DO, _DX)] = x_ref[:, pl.ds(l * _DX, _DX)]
            o_ref[:, pl.ds(l * _DO + _DX, _DE)] = jnp.broadcast_to(
                tbl_ref[pl.ds(l * _DE, _DE)][None], (_BT, _DE))

    return pl.pallas_call(
        body,
        grid=(_B // _BT,),
        in_specs=[
            pl.BlockSpec((_BT, _XW), lambda i: (i, 0)),
            pl.BlockSpec((_LPAD * _DE,), lambda i: (0,)),
        ],
        out_specs=pl.BlockSpec((_BT, _OW), lambda i: (i, 0)),
        out_shape=jax.ShapeDtypeStruct((_B, _OW), jnp.float32),
    )(xf, tblf)


def kernel(x, emb):
    tbl = _sc_lookup(emb)
    out = _tc_concat(x.reshape(_B, _XW), tbl.reshape(_LPAD * _DE))
    return out.reshape(_B, _L, _DO)
